# baseline (speedup 1.0000x reference)
import functools

import jax
import jax.numpy as jnp
from jax import lax
from jax.experimental import pallas as pl
from jax.experimental.pallas import tpu as pltpu
from jax.experimental.pallas import tpu_sc as plsc

C = 16
G = 128
V = G * G * G
N = 1048576

_NC = 2
_NS = 16
_NW = _NC * _NS
_BPW = N // _NW
_CH = 2048
_NCH = _BPW // _CH
_IPG = 128
_GPC = _CH // _IPG


def _snap(t):
    c = t.astype(jnp.int32)
    f = t - c.astype(jnp.float32)
    up = (f > 0.5) | ((f == 0.5) & ((c & 1) == 1))
    return c + jnp.where(up, 1, 0)


def _gather_all(coords, table):
    mesh = plsc.VectorSubcoreMesh(core_axis_name="c", subcore_axis_name="s")

    @functools.partial(
        pl.kernel,
        mesh=mesh,
        compiler_params=pltpu.CompilerParams(
            use_tc_tiling_on_sc=False, needs_layout_passes=False),
        out_type=jax.ShapeDtypeStruct((N, C), jnp.float32),
        scratch_types=[
            pltpu.VMEM((2, _CH, 3), jnp.float32),
            pltpu.VMEM((2, _GPC, _IPG), jnp.int32),
            pltpu.VMEM((2, _CH, C), jnp.float32),
            pltpu.SemaphoreType.DMA,
            pltpu.SemaphoreType.DMA,
            pltpu.SemaphoreType.DMA,
            pltpu.SemaphoreType.DMA,
            pltpu.SemaphoreType.DMA,
            pltpu.SemaphoreType.DMA,
        ],
    )
    def body(coords_hbm, table_hbm, out_hbm,
             cv, idx_v, rows_v, cs0, cs1, gs0, gs1, ws0, ws1):
        wid = lax.axis_index("s") * _NC + lax.axis_index("c")
        base = wid * _BPW
        csem = (cs0, cs1)
        gsem = (gs0, gs1)
        wsem = (ws0, ws1)
        lane = lax.iota(jnp.int32, 16)
        col0 = jnp.zeros((16,), jnp.int32)
        col1 = col0 + 1
        col2 = col0 + 2

        def compute_idx(p):
            def step2(i, carry):
                rows = i * 16 + lane
                x = plsc.load_gather(cv.at[p], [rows, col0])
                y = plsc.load_gather(cv.at[p], [rows, col1])
                z = plsc.load_gather(cv.at[p], [rows, col2])
                ix = _snap(x * (G - 1.0))
                iy = _snap(y * (G - 1.0))
                iz = _snap(z * (G - 1.0))
                flat = (ix * G + iy) * G + iz
                j = i // 8
                k = i - j * 8
                idx_v[p, j, pl.ds(k * 16, 16)] = flat
                return carry
            lax.fori_loop(0, _CH // 16, step2, 0)

        def copy_coords(g, p):
            off = base + g * _CH
            return pltpu.async_copy(
                coords_hbm.at[pl.ds(off, _CH)], cv.at[p], csem[p])

        def fire_gathers(p):
            return [
                pltpu.async_copy(
                    table_hbm.at[idx_v.at[p, j]],
                    rows_v.at[p, pl.ds(j * _IPG, _IPG)],
                    gsem[p])
                for j in range(_GPC)
            ]

        def writeback(g, p):
            off = base + g * _CH
            return pltpu.async_copy(
                rows_v.at[p], out_hbm.at[pl.ds(off, _CH)], wsem[p])

        gh = [None, None]
        wb = [None, None]
        copy_coords(0, 0).wait()
        for g in range(_NCH):
            p = g & 1
            if g + 1 < _NCH:
                cnext = copy_coords(g + 1, 1 - p)
            compute_idx(p)
            if wb[p] is not None:
                wb[p].wait()
            if gh[1 - p] is not None:
                for h in gh[1 - p]:
                    h.wait()
                wb[1 - p] = writeback(g - 1, 1 - p)
            gh[p] = fire_gathers(p)
            if g + 1 < _NCH:
                cnext.wait()
        p = (_NCH - 1) & 1
        for h in gh[p]:
            h.wait()
        wb[p] = writeback(_NCH - 1, p)
        wb[0].wait()
        wb[1].wait()

    return body(coords, table)


def kernel(coords, grid):
    table = grid.reshape(C, V).T
    return _gather_all(coords, table)


# speedup vs baseline: 1.1248x; 1.1248x over previous
import functools

import jax
import jax.numpy as jnp
from jax import lax
from jax.experimental import pallas as pl
from jax.experimental.pallas import tpu as pltpu
from jax.experimental.pallas import tpu_sc as plsc

C = 16
G = 128
V = G * G * G
N = 1048576

_NC = 2
_NS = 16
_NW = _NC * _NS
_BPW = N // _NW
_CH = 2048
_NCH = _BPW // _CH
_IPG = 128
_GPC = _CH // _IPG


def _snap(t):
    c = t.astype(jnp.int32)
    f = t - c.astype(jnp.float32)
    up = (f > 0.5) | ((f == 0.5) & ((c & 1) == 1))
    return c + jnp.where(up, 1, 0)


def _gather_all(coords, table):
    mesh = plsc.VectorSubcoreMesh(core_axis_name="c", subcore_axis_name="s")

    @functools.partial(
        pl.kernel,
        mesh=mesh,
        compiler_params=pltpu.CompilerParams(
            use_tc_tiling_on_sc=False, needs_layout_passes=False),
        out_type=jax.ShapeDtypeStruct((N, C), jnp.float32),
        scratch_types=[
            pltpu.VMEM((2, 3 * _CH), jnp.float32),
            pltpu.VMEM((2, _GPC, _IPG), jnp.int32),
            pltpu.VMEM((2, _CH, C), jnp.float32),
            pltpu.SemaphoreType.DMA,
            pltpu.SemaphoreType.DMA,
            pltpu.SemaphoreType.DMA,
            pltpu.SemaphoreType.DMA,
            pltpu.SemaphoreType.DMA,
            pltpu.SemaphoreType.DMA,
        ],
    )
    def body(coords_hbm, table_hbm, out_hbm,
             cv, idx_v, rows_v, cs0, cs1, gs0, gs1, ws0, ws1):
        wid = lax.axis_index("s") * _NC + lax.axis_index("c")
        base = wid * _BPW
        csem = (cs0, cs1)
        gsem = (gs0, gs1)
        wsem = (ws0, ws1)
        lane = lax.iota(jnp.int32, 16)
        lane3 = lane * 3

        def compute_idx(p):
            def step2(i, carry):
                b = i * 48 + lane3
                x = plsc.load_gather(cv.at[p], [b])
                y = plsc.load_gather(cv.at[p], [b + 1])
                z = plsc.load_gather(cv.at[p], [b + 2])
                ix = _snap(x * (G - 1.0))
                iy = _snap(y * (G - 1.0))
                iz = _snap(z * (G - 1.0))
                flat = (ix * G + iy) * G + iz
                j = i // 8
                k = i - j * 8
                idx_v[p, j, pl.ds(k * 16, 16)] = flat
                return carry
            lax.fori_loop(0, _CH // 16, step2, 0)

        def copy_coords(g, p):
            off = 3 * (base + g * _CH)
            return pltpu.async_copy(
                coords_hbm.at[pl.ds(off, 3 * _CH)], cv.at[p], csem[p])

        def fire_gathers(p):
            return [
                pltpu.async_copy(
                    table_hbm.at[idx_v.at[p, j]],
                    rows_v.at[p, pl.ds(j * _IPG, _IPG)],
                    gsem[p])
                for j in range(_GPC)
            ]

        def writeback(g, p):
            off = base + g * _CH
            return pltpu.async_copy(
                rows_v.at[p], out_hbm.at[pl.ds(off, _CH)], wsem[p])

        gh = [None, None]
        wb = [None, None]
        copy_coords(0, 0).wait()
        for g in range(_NCH):
            p = g & 1
            if g + 1 < _NCH:
                cnext = copy_coords(g + 1, 1 - p)
            compute_idx(p)
            if wb[p] is not None:
                wb[p].wait()
            if gh[1 - p] is not None:
                for h in gh[1 - p]:
                    h.wait()
                wb[1 - p] = writeback(g - 1, 1 - p)
            plsc.subcore_barrier()
            gh[p] = fire_gathers(p)
            if g + 1 < _NCH:
                cnext.wait()
        p = (_NCH - 1) & 1
        for h in gh[p]:
            h.wait()
        wb[p] = writeback(_NCH - 1, p)
        wb[0].wait()
        wb[1].wait()

    return body(coords, table)


def kernel(coords, grid):
    table = grid.reshape(C, V).T
    coords_flat = coords.reshape(3 * N)
    return _gather_all(coords_flat, table)


# speedup vs baseline: 2.2429x; 1.9941x over previous
import functools

import jax
import jax.numpy as jnp
from jax import lax
from jax.experimental import pallas as pl
from jax.experimental.pallas import tpu as pltpu
from jax.experimental.pallas import tpu_sc as plsc

C = 16
G = 128
V = G * G * G
N = 1048576


_BS = 131072


def _snap(t):
    w = t.astype(jnp.int32)
    f = t - w.astype(jnp.float32)
    up = (f > 0.5) | ((f == 0.5) & ((w & 1) == 1))
    return w + jnp.where(up, 1, 0)


def _idx_body(x_ref, y_ref, z_ref, o_ref):
    ix = _snap(x_ref[...] * (G - 1.0))
    iy = _snap(y_ref[...] * (G - 1.0))
    iz = _snap(z_ref[...] * (G - 1.0))
    o_ref[...] = (ix * G + iy) * G + iz


def _compute_indices(cx, cy, cz):
    spec = pl.BlockSpec((_BS,), lambda i: (i,))
    return pl.pallas_call(
        _idx_body,
        grid=(N // _BS,),
        in_specs=[spec, spec, spec],
        out_specs=spec,
        out_shape=jax.ShapeDtypeStruct((N,), jnp.int32),
    )(cx, cy, cz)


_NC = 2
_NS = 16
_NW = _NC * _NS
_BPW = N // _NW
_CH = 2048
_NCH = _BPW // _CH
_IPG = 128
_GPC = _CH // _IPG


def _gather_all(idx, table):
    mesh = plsc.VectorSubcoreMesh(core_axis_name="c", subcore_axis_name="s")

    @functools.partial(
        pl.kernel,
        mesh=mesh,
        compiler_params=pltpu.CompilerParams(
            use_tc_tiling_on_sc=False, needs_layout_passes=False),
        out_type=jax.ShapeDtypeStruct((N, C), jnp.float32),
        scratch_types=[
            pltpu.VMEM((2, _CH), jnp.int32),
            pltpu.VMEM((2, _CH, C), jnp.float32),
            pltpu.SemaphoreType.DMA,
            pltpu.SemaphoreType.DMA,
            pltpu.SemaphoreType.DMA,
            pltpu.SemaphoreType.DMA,
            pltpu.SemaphoreType.DMA,
            pltpu.SemaphoreType.DMA,
        ],
    )
    def body(idx_hbm, table_hbm, out_hbm,
             idx_v, rows_v, cs0, cs1, gs0, gs1, ws0, ws1):
        wid = lax.axis_index("s") * _NC + lax.axis_index("c")
        base = wid * _BPW
        csem = (cs0, cs1)
        gsem = (gs0, gs1)
        wsem = (ws0, ws1)

        def copy_idx(g, p):
            off = base + g * _CH
            return pltpu.async_copy(
                idx_hbm.at[pl.ds(off, _CH)], idx_v.at[p], csem[p])

        def fire_gathers(p):
            return [
                pltpu.async_copy(
                    table_hbm.at[idx_v.at[p, pl.ds(j * _IPG, _IPG)]],
                    rows_v.at[p, pl.ds(j * _IPG, _IPG)],
                    gsem[p])
                for j in range(_GPC)
            ]

        def writeback(g, p):
            off = base + g * _CH
            return pltpu.async_copy(
                rows_v.at[p], out_hbm.at[pl.ds(off, _CH)], wsem[p])

        gh = [None, None]
        wb = [None, None]
        copy_idx(0, 0).wait()
        for g in range(_NCH):
            p = g & 1
            if g + 1 < _NCH:
                cnext = copy_idx(g + 1, 1 - p)
            if wb[p] is not None:
                wb[p].wait()
            if gh[1 - p] is not None:
                for h in gh[1 - p]:
                    h.wait()
                wb[1 - p] = writeback(g - 1, 1 - p)
            gh[p] = fire_gathers(p)
            if g + 1 < _NCH:
                cnext.wait()
        p = (_NCH - 1) & 1
        for h in gh[p]:
            h.wait()
        wb[p] = writeback(_NCH - 1, p)
        wb[0].wait()
        wb[1].wait()

    return body(idx, table)


def kernel(coords, grid):
    table = jnp.moveaxis(grid, 0, -1).reshape(V, C)
    idx = _compute_indices(coords[:, 0], coords[:, 1], coords[:, 2])
    return _gather_all(idx, table)


# speedup vs baseline: 2.8367x; 1.2648x over previous
import functools

import jax
import jax.numpy as jnp
from jax import lax
from jax.experimental import pallas as pl
from jax.experimental.pallas import tpu as pltpu
from jax.experimental.pallas import tpu_sc as plsc

C = 16
G = 128
V = G * G * G
N = 1048576


_BS = 131072


def _snap(t):
    w = t.astype(jnp.int32)
    f = t - w.astype(jnp.float32)
    up = (f > 0.5) | ((f == 0.5) & ((w & 1) == 1))
    return w + jnp.where(up, 1, 0)


def _idx_body(x_ref, y_ref, z_ref, o_ref):
    ix = _snap(x_ref[...] * (G - 1.0))
    iy = _snap(y_ref[...] * (G - 1.0))
    iz = _snap(z_ref[...] * (G - 1.0))
    o_ref[...] = (ix * G + iy) * G + iz


def _compute_indices(cx, cy, cz):
    spec = pl.BlockSpec((_BS,), lambda i: (i,))
    return pl.pallas_call(
        _idx_body,
        grid=(N // _BS,),
        in_specs=[spec, spec, spec],
        out_specs=spec,
        out_shape=jax.ShapeDtypeStruct((N,), jnp.int32),
    )(cx, cy, cz)


_NC = 2
_NS = 16
_NW = _NC * _NS
_BPW = N // _NW
_CH = 1024
_NCH = _BPW // _CH
_IPG = 128
_GPC = _CH // _IPG
_NB = N // 128


def _gather_all(idx, table):
    mesh = plsc.VectorSubcoreMesh(core_axis_name="c", subcore_axis_name="s")

    @functools.partial(
        pl.kernel,
        mesh=mesh,
        compiler_params=pltpu.CompilerParams(
            use_tc_tiling_on_sc=False, needs_layout_passes=False),
        out_type=jax.ShapeDtypeStruct((2, _NB, 8, 128), jnp.float32),
        scratch_types=[
            pltpu.VMEM((2, _CH), jnp.int32),
            pltpu.VMEM((2, _CH, C), jnp.float32),
            pltpu.VMEM((2, 2, _CH // 128, 8, 128), jnp.float32),
            pltpu.SemaphoreType.DMA,
            pltpu.SemaphoreType.DMA,
            pltpu.SemaphoreType.DMA,
            pltpu.SemaphoreType.DMA,
            pltpu.SemaphoreType.DMA,
            pltpu.SemaphoreType.DMA,
        ],
    )
    def body(idx_hbm, table_hbm, out_hbm,
             idx_v, rows_v, tr_v, cs0, cs1, gs0, gs1, ws0, ws1):
        wid = lax.axis_index("s") * _NC + lax.axis_index("c")
        base = wid * _BPW
        csem = (cs0, cs1)
        gsem = (gs0, gs1)
        wsem = (ws0, ws1)
        lane = lax.iota(jnp.int32, 16)

        def copy_idx(g, p):
            off = base + g * _CH
            return pltpu.async_copy(
                idx_hbm.at[pl.ds(off, _CH)], idx_v.at[p], csem[p])

        def fire_gathers(p):
            return [
                pltpu.async_copy(
                    table_hbm.at[idx_v.at[p, pl.ds(j * _IPG, _IPG)]],
                    rows_v.at[p, pl.ds(j * _IPG, _IPG)],
                    gsem[p])
                for j in range(_GPC)
            ]

        def transpose(p):
            def step(i, carry):
                b = i >> 4
                hc = i & 15
                h = hc >> 3
                c = hc & 7
                col = hc + jnp.zeros((16,), jnp.int32)
                for lg in range(8):
                    row = b * 128 + lg * 16 + lane
                    val = plsc.load_gather(rows_v.at[p], [row, col])
                    tr_v[p, h, b, c, pl.ds(lg * 16, 16)] = val
                return carry
            lax.fori_loop(0, (_CH // 128) * 16, step, 0)

        def writeback(g, p):
            blk0 = (base + g * _CH) // 128
            return [
                pltpu.async_copy(
                    tr_v.at[p, h],
                    out_hbm.at[h, pl.ds(blk0, _CH // 128)], wsem[p])
                for h in range(2)
            ]

        gh = [None, None]
        wb = [None, None]
        copy_idx(0, 0).wait()
        for g in range(_NCH):
            p = g & 1
            if gh[1 - p] is not None:
                for h in gh[1 - p]:
                    h.wait()
            if g + 1 < _NCH:
                cnext = copy_idx(g + 1, 1 - p)
            gh[p] = fire_gathers(p)
            if g >= 1:
                if wb[1 - p] is not None:
                    for h in wb[1 - p]:
                        h.wait()
                transpose(1 - p)
                plsc.subcore_barrier()
                wb[1 - p] = writeback(g - 1, 1 - p)
            if g + 1 < _NCH:
                cnext.wait()
        p = (_NCH - 1) & 1
        for h in gh[p]:
            h.wait()
        if wb[p] is not None:
            for h in wb[p]:
                h.wait()
        transpose(p)
        plsc.subcore_barrier()
        wb[p] = writeback(_NCH - 1, p)
        for h in wb[0] + wb[1]:
            h.wait()

    return body(idx, table)


def kernel(coords, grid):
    table = jnp.moveaxis(grid, 0, -1).reshape(V, C)
    idx = _compute_indices(coords[:, 0], coords[:, 1], coords[:, 2])
    res = _gather_all(idx, table)
    return res.transpose(1, 3, 0, 2).reshape(N, C)
